# baseline (speedup 1.0000x reference)
import functools

import jax
import jax.numpy as jnp
from jax import lax
from jax.experimental import pallas as pl
from jax.experimental.pallas import tpu as pltpu
from jax.experimental.pallas import tpu_sc as plsc

NUM_USERS = 100000
NUM_ITEMS = 100000
EMBED_DIM = 64
BATCH = 16384

_NC = 2
_NS = 16
_NW = _NC * _NS
_BPW = BATCH // _NW
_CHUNK = 128
_NCHUNK = _BPW // _CHUNK
_GPC = _CHUNK // 16


def _sc_body(uid_hbm, iid_hbm, ue2_hbm, ie2_hbm, ug_hbm, ig_hbm,
             dot_hbm, s_hbm,
             uid_v, iid_v, hu_v, hi_v,
             ue_b0, ue_b1, ie_b0, ie_b1,
             ug_v, ig_v, dot_v, s_v,
             sem_u0, sem_u1, sem_i0, sem_i1, sem_ug, sem_ig):
    wid = lax.axis_index("s") * _NC + lax.axis_index("c")
    base = wid * _BPW

    pltpu.sync_copy(uid_hbm.at[pl.ds(base, _BPW)], uid_v)
    pltpu.sync_copy(iid_hbm.at[pl.ds(base, _BPW)], iid_v)

    cp_ug = pltpu.async_copy(ug_hbm.at[uid_v], ug_v, sem_ug)
    cp_ig = pltpu.async_copy(ig_hbm.at[iid_v], ig_v, sem_ig)

    lane = jnp.arange(16, dtype=jnp.int32)

    ue_bufs = (ue_b0, ue_b1)
    ie_bufs = (ie_b0, ie_b1)
    sems_u = (sem_u0, sem_u1)
    sems_i = (sem_i0, sem_i1)

    def fire(c):
        sl = pl.ds(c * _CHUNK, _CHUNK)
        cu = pltpu.async_copy(ue2_hbm.at[uid_v.at[sl]], ue_bufs[c % 2], sems_u[c % 2])
        ci = pltpu.async_copy(ie2_hbm.at[iid_v.at[sl]], ie_bufs[c % 2], sems_i[c % 2])
        return cu, ci

    pend = fire(0)
    for c in range(_NCHUNK):
        pend[0].wait()
        pend[1].wait()
        if c + 1 < _NCHUNK:
            nxt = fire(c + 1)
        ue_v = ue_bufs[c % 2]
        ie_v = ie_bufs[c % 2]
        cbase = c * _CHUNK

        def group(g, _):
            r0 = cbase + g * 16
            row = lane + g * 16
            acc = jnp.zeros((16,), jnp.float32)
            for j in range(EMBED_DIM):
                d = (lane + j) & (EMBED_DIM - 1)
                u = plsc.load_gather(ue_v, [row, d])
                v = plsc.load_gather(ie_v, [row, d + EMBED_DIM])
                acc = acc + u * v
            dot_v[pl.ds(r0, 16)] = acc
            return _

        lax.fori_loop(0, _GPC, group, None)
        if c + 1 < _NCHUNK:
            pend = nxt

    cp_ug.wait()
    cp_ig.wait()

    def gsum(g, _):
        r0 = g * 16
        s_v[pl.ds(r0, 16)] = ug_v[pl.ds(r0, 16)] + ig_v[pl.ds(r0, 16)]
        return _
    lax.fori_loop(0, _BPW // 16, gsum, None)

    pltpu.sync_copy(dot_v, dot_hbm.at[pl.ds(base, _BPW)])
    pltpu.sync_copy(s_v, s_hbm.at[pl.ds(base, _BPW)])


@jax.jit
def _sc_call(uid, iid, ue2, ie2, ug, ig):
    mesh = plsc.VectorSubcoreMesh(core_axis_name="c", subcore_axis_name="s")
    f = functools.partial(
        pl.kernel, _sc_body, mesh=mesh,
        compiler_params=pltpu.CompilerParams(
            needs_layout_passes=False, use_tc_tiling_on_sc=False),
        out_type=[
            jax.ShapeDtypeStruct((BATCH,), jnp.float32),
            jax.ShapeDtypeStruct((BATCH,), jnp.float32),
        ],
        scratch_types=[
            pltpu.VMEM((_BPW,), jnp.int32),
            pltpu.VMEM((_BPW,), jnp.int32),
            pltpu.VMEM((_BPW,), jnp.int32),
            pltpu.VMEM((_BPW,), jnp.int32),
            pltpu.VMEM((_CHUNK, 2 * EMBED_DIM), jnp.float32),
            pltpu.VMEM((_CHUNK, 2 * EMBED_DIM), jnp.float32),
            pltpu.VMEM((_CHUNK, 2 * EMBED_DIM), jnp.float32),
            pltpu.VMEM((_CHUNK, 2 * EMBED_DIM), jnp.float32),
            pltpu.VMEM((_BPW,), jnp.float32),
            pltpu.VMEM((_BPW,), jnp.float32),
            pltpu.VMEM((_BPW,), jnp.float32),
            pltpu.VMEM((_BPW,), jnp.float32),
            pltpu.SemaphoreType.DMA,
            pltpu.SemaphoreType.DMA,
            pltpu.SemaphoreType.DMA,
            pltpu.SemaphoreType.DMA,
            pltpu.SemaphoreType.DMA,
            pltpu.SemaphoreType.DMA,
        ],
    )()
    return f(uid, iid, ue2, ie2, ug, ig)


_PACK_IN_BLK = 2048
_PACK_BLOCKS = (NUM_USERS + _PACK_IN_BLK - 1) // _PACK_IN_BLK
_PACK_ROWS = _PACK_BLOCKS * _PACK_IN_BLK // 2


def _tc_pack_body(xu_ref, xi_ref, ou_ref, oi_ref):
    ey = jnp.eye(EMBED_DIM, dtype=jnp.float32)
    dn = (((0,), (0,)), ((), ()))
    cu = jax.lax.dot_general(xu_ref[...], ey, dn,
                             preferred_element_type=jnp.float32)
    ci = jax.lax.dot_general(xi_ref[...], ey, dn,
                             preferred_element_type=jnp.float32)
    for k in range(_PACK_IN_BLK // 512):
        q0 = 256 * k
        r0 = 512 * k
        ou_ref[q0:q0 + 256, 0:EMBED_DIM] = cu[r0:r0 + 256]
        ou_ref[q0:q0 + 256, EMBED_DIM:2 * EMBED_DIM] = cu[r0 + 256:r0 + 512]
        oi_ref[q0:q0 + 256, 0:EMBED_DIM] = ci[r0:r0 + 256]
        oi_ref[q0:q0 + 256, EMBED_DIM:2 * EMBED_DIM] = ci[r0 + 256:r0 + 512]


@jax.jit
def _tc_pack(te_u, te_i):
    return pl.pallas_call(
        _tc_pack_body,
        grid=(_PACK_BLOCKS,),
        in_specs=[
            pl.BlockSpec((EMBED_DIM, _PACK_IN_BLK), lambda i: (0, i)),
            pl.BlockSpec((EMBED_DIM, _PACK_IN_BLK), lambda i: (0, i)),
        ],
        out_specs=[
            pl.BlockSpec((_PACK_IN_BLK // 2, 2 * EMBED_DIM), lambda i: (i, 0)),
            pl.BlockSpec((_PACK_IN_BLK // 2, 2 * EMBED_DIM), lambda i: (i, 0)),
        ],
        out_shape=[
            jax.ShapeDtypeStruct((_PACK_ROWS, 2 * EMBED_DIM), jnp.float32),
            jax.ShapeDtypeStruct((_PACK_ROWS, 2 * EMBED_DIM), jnp.float32),
        ],
    )(te_u, te_i)


def _tc_softplus_body(s_ref, o_ref):
    o_ref[...] = jax.nn.softplus(s_ref[...])


@jax.jit
def _tc_softplus(s2d):
    return pl.pallas_call(
        _tc_softplus_body,
        out_shape=jax.ShapeDtypeStruct(s2d.shape, s2d.dtype),
    )(s2d)


def kernel(user_ids, item_ids, user_emb, item_emb, user_gamma, item_gamma):
    uid = user_ids.astype(jnp.int32)
    iid = item_ids.astype(jnp.int32)
    cat = jnp.concatenate([user_emb, item_emb], axis=1)
    ue2 = cat
    ie2 = cat
    ug1 = user_gamma.reshape(NUM_USERS)
    ig1 = item_gamma.reshape(NUM_ITEMS)
    dot, s = _sc_call(uid, iid, ue2, ie2, ug1, ig1)
    var = _tc_softplus(s.reshape(128, 128)).reshape(BATCH)
    return (dot, var)


# speedup vs baseline: 1.5033x; 1.5033x over previous
import functools

import jax
import jax.numpy as jnp
from jax import lax
from jax.experimental import pallas as pl
from jax.experimental.pallas import tpu as pltpu
from jax.experimental.pallas import tpu_sc as plsc

NUM_USERS = 100000
NUM_ITEMS = 100000
EMBED_DIM = 64
BATCH = 16384

_NC = 2
_NS = 16
_NW = _NC * _NS
_BPW = BATCH // _NW
_CHUNK = 128
_NCHUNK = _BPW // _CHUNK
_GPC = _CHUNK // 16


def _sc_body(uid_hbm, iid_hbm, ue2_hbm, ie2_hbm, ug_hbm, ig_hbm,
             dot_hbm, s_hbm,
             uid_v, iid_v, hu_v, hi_v,
             ue_b0, ue_b1, ie_b0, ie_b1,
             ug_v, ig_v, dot_v, s_v,
             sem_u0, sem_u1, sem_i0, sem_i1, sem_ug, sem_ig):
    wid = lax.axis_index("s") * _NC + lax.axis_index("c")
    base = wid * _BPW

    pltpu.sync_copy(uid_hbm.at[pl.ds(base, _BPW)], uid_v)
    pltpu.sync_copy(iid_hbm.at[pl.ds(base, _BPW)], iid_v)

    cp_ug = pltpu.async_copy(ug_hbm.at[uid_v], ug_v, sem_ug)
    cp_ig = pltpu.async_copy(ig_hbm.at[iid_v], ig_v, sem_ig)

    lane = jnp.arange(16, dtype=jnp.int32)

    def mkidx(g, _):
        r0 = g * 16
        u = uid_v[pl.ds(r0, 16)]
        i = iid_v[pl.ds(r0, 16)]
        hu_v[pl.ds(r0, 16)] = ((u >> 9) << 8) | (u & 255)
        hi_v[pl.ds(r0, 16)] = ((i >> 9) << 8) | (i & 255)
        return _
    lax.fori_loop(0, _BPW // 16, mkidx, None)

    ue_bufs = (ue_b0, ue_b1)
    ie_bufs = (ie_b0, ie_b1)
    sems_u = (sem_u0, sem_u1)
    sems_i = (sem_i0, sem_i1)

    def fire(c):
        sl = pl.ds(c * _CHUNK, _CHUNK)
        cu = pltpu.async_copy(ue2_hbm.at[hu_v.at[sl]], ue_bufs[c % 2], sems_u[c % 2])
        ci = pltpu.async_copy(ie2_hbm.at[hi_v.at[sl]], ie_bufs[c % 2], sems_i[c % 2])
        return cu, ci

    pend = fire(0)
    for c in range(_NCHUNK):
        pend[0].wait()
        pend[1].wait()
        if c + 1 < _NCHUNK:
            nxt = fire(c + 1)
        ue_v = ue_bufs[c % 2]
        ie_v = ie_bufs[c % 2]
        cbase = c * _CHUNK

        def group(g, _):
            r0 = cbase + g * 16
            u16 = uid_v[pl.ds(r0, 16)]
            i16 = iid_v[pl.ds(r0, 16)]
            pu = ((u16 >> 8) & 1) << 6
            pi = ((i16 >> 8) & 1) << 6
            row = lane + g * 16
            acc = jnp.zeros((16,), jnp.float32)
            for j in range(EMBED_DIM):
                d = (lane + j) & (EMBED_DIM - 1)
                u = plsc.load_gather(ue_v, [row, d + pu])
                v = plsc.load_gather(ie_v, [row, d + pi])
                acc = acc + u * v
            dot_v[pl.ds(r0, 16)] = acc
            return _

        lax.fori_loop(0, _GPC, group, None)
        if c + 1 < _NCHUNK:
            pend = nxt

    cp_ug.wait()
    cp_ig.wait()

    def gsum(g, _):
        r0 = g * 16
        s_v[pl.ds(r0, 16)] = ug_v[pl.ds(r0, 16)] + ig_v[pl.ds(r0, 16)]
        return _
    lax.fori_loop(0, _BPW // 16, gsum, None)

    pltpu.sync_copy(dot_v, dot_hbm.at[pl.ds(base, _BPW)])
    pltpu.sync_copy(s_v, s_hbm.at[pl.ds(base, _BPW)])


@jax.jit
def _sc_call(uid, iid, ue2, ie2, ug, ig):
    mesh = plsc.VectorSubcoreMesh(core_axis_name="c", subcore_axis_name="s")
    f = functools.partial(
        pl.kernel, _sc_body, mesh=mesh,
        compiler_params=pltpu.CompilerParams(
            needs_layout_passes=False, use_tc_tiling_on_sc=False),
        out_type=[
            jax.ShapeDtypeStruct((BATCH,), jnp.float32),
            jax.ShapeDtypeStruct((BATCH,), jnp.float32),
        ],
        scratch_types=[
            pltpu.VMEM((_BPW,), jnp.int32),
            pltpu.VMEM((_BPW,), jnp.int32),
            pltpu.VMEM((_BPW,), jnp.int32),
            pltpu.VMEM((_BPW,), jnp.int32),
            pltpu.VMEM((_CHUNK, 2 * EMBED_DIM), jnp.float32),
            pltpu.VMEM((_CHUNK, 2 * EMBED_DIM), jnp.float32),
            pltpu.VMEM((_CHUNK, 2 * EMBED_DIM), jnp.float32),
            pltpu.VMEM((_CHUNK, 2 * EMBED_DIM), jnp.float32),
            pltpu.VMEM((_BPW,), jnp.float32),
            pltpu.VMEM((_BPW,), jnp.float32),
            pltpu.VMEM((_BPW,), jnp.float32),
            pltpu.VMEM((_BPW,), jnp.float32),
            pltpu.SemaphoreType.DMA,
            pltpu.SemaphoreType.DMA,
            pltpu.SemaphoreType.DMA,
            pltpu.SemaphoreType.DMA,
            pltpu.SemaphoreType.DMA,
            pltpu.SemaphoreType.DMA,
        ],
    )()
    return f(uid, iid, ue2, ie2, ug, ig)


_PACK_IN_BLK = 4096
_PACK_BLOCKS = (NUM_USERS + _PACK_IN_BLK - 1) // _PACK_IN_BLK
_PACK_ROWS = _PACK_BLOCKS * _PACK_IN_BLK // 2


def _tc_pack_body(xu_ref, xi_ref, gu_ref, gi_ref, ou_ref, oi_ref,
                  ogu_ref, ogi_ref):
    ogu_ref[...] = gu_ref[0, :]
    ogi_ref[...] = gi_ref[0, :]
    ey = jnp.eye(EMBED_DIM, dtype=jnp.float32)
    dn = (((0,), (0,)), ((), ()))
    cu = jax.lax.dot_general(xu_ref[...], ey, dn,
                             preferred_element_type=jnp.float32)
    ci = jax.lax.dot_general(xi_ref[...], ey, dn,
                             preferred_element_type=jnp.float32)
    for k in range(_PACK_IN_BLK // 512):
        q0 = 256 * k
        r0 = 512 * k
        ou_ref[q0:q0 + 256, 0:EMBED_DIM] = cu[r0:r0 + 256]
        ou_ref[q0:q0 + 256, EMBED_DIM:2 * EMBED_DIM] = cu[r0 + 256:r0 + 512]
        oi_ref[q0:q0 + 256, 0:EMBED_DIM] = ci[r0:r0 + 256]
        oi_ref[q0:q0 + 256, EMBED_DIM:2 * EMBED_DIM] = ci[r0 + 256:r0 + 512]


@jax.jit
def _tc_pack(te_u, te_i, g_u, g_i):
    return pl.pallas_call(
        _tc_pack_body,
        grid=(_PACK_BLOCKS,),
        in_specs=[
            pl.BlockSpec((EMBED_DIM, _PACK_IN_BLK), lambda i: (0, i)),
            pl.BlockSpec((EMBED_DIM, _PACK_IN_BLK), lambda i: (0, i)),
            pl.BlockSpec((1, _PACK_IN_BLK), lambda i: (0, i)),
            pl.BlockSpec((1, _PACK_IN_BLK), lambda i: (0, i)),
        ],
        out_specs=[
            pl.BlockSpec((_PACK_IN_BLK // 2, 2 * EMBED_DIM), lambda i: (i, 0)),
            pl.BlockSpec((_PACK_IN_BLK // 2, 2 * EMBED_DIM), lambda i: (i, 0)),
            pl.BlockSpec((_PACK_IN_BLK,), lambda i: (i,)),
            pl.BlockSpec((_PACK_IN_BLK,), lambda i: (i,)),
        ],
        out_shape=[
            jax.ShapeDtypeStruct((_PACK_ROWS, 2 * EMBED_DIM), jnp.float32),
            jax.ShapeDtypeStruct((_PACK_ROWS, 2 * EMBED_DIM), jnp.float32),
            jax.ShapeDtypeStruct((2 * _PACK_ROWS,), jnp.float32),
            jax.ShapeDtypeStruct((2 * _PACK_ROWS,), jnp.float32),
        ],
    )(te_u, te_i, g_u, g_i)


def _tc_softplus_body(s_ref, o_ref):
    o_ref[...] = jax.nn.softplus(s_ref[...])


@jax.jit
def _tc_softplus(s2d):
    return pl.pallas_call(
        _tc_softplus_body,
        out_shape=jax.ShapeDtypeStruct(s2d.shape, s2d.dtype),
    )(s2d)


def kernel(user_ids, item_ids, user_emb, item_emb, user_gamma, item_gamma):
    uid = user_ids.astype(jnp.int32)
    iid = item_ids.astype(jnp.int32)
    ue2, ie2, ug1, ig1 = _tc_pack(user_emb.T, item_emb.T,
                                  user_gamma.T, item_gamma.T)
    dot, s = _sc_call(uid, iid, ue2, ie2, ug1, ig1)
    var = _tc_softplus(s.reshape(128, 128)).reshape(BATCH)
    return (dot, var)
